# baseline (speedup 1.0000x reference)
import functools

import jax
import jax.numpy as jnp
from jax import lax
from jax.experimental import pallas as pl
from jax.experimental.pallas import tpu as pltpu
from jax.experimental.pallas import tpu_sc as plsc


def _build_wext_kernel(wr_ref, out_ref):
    f32 = jnp.float32
    w = wr_ref[:]
    ri = jax.lax.broadcasted_iota(jnp.int32, (128, 128), 0)
    ci = jax.lax.broadcasted_iota(jnp.int32, (128, 128), 1)

    def dot(a, b):
        return jax.lax.dot(a, b, preferred_element_type=f32,
                           precision=jax.lax.Precision.HIGHEST)

    perm = (ri == (8 * (15 - ci // 8) + ci % 8)).astype(f32)
    flip = ((ri + ci) == 127).astype(f32)
    rev = dot(flip, dot(w, perm))
    roll8 = (ri == ((ci + 8) % 128)).astype(f32)
    gfw = dot(w, roll8)
    pick = ((ri == 127) & (ci == 0)).astype(f32)
    b0 = dot(pick, gfw)
    low = jnp.where((ri < 127) | (ci < 120), rev, b0)
    gup = jnp.concatenate([gfw[1:], jnp.zeros((1, 128), f32)], axis=0)
    high = jnp.where(ci < 120, gfw, gup)
    out_ref[:] = jnp.concatenate([low, high], axis=0)


def _sc_expand_kernel(wext_hbm, out_hbm, wext_v, sem):
    cid = lax.axis_index("c")
    sid = lax.axis_index("s")
    wid = sid * 2 + cid
    pltpu.sync_copy(wext_hbm, wext_v)

    def body(k, carry):
        u = wid * 64 + k
        off = (2047 - u) * 8
        pltpu.async_copy(
            wext_v.at[pl.ds(off, 16384)],
            out_hbm.at[pl.ds(u * 16384, 16384)], sem).wait()
        return carry

    lax.fori_loop(0, 64, body, 0)


def kernel(x, W):
    bs, _, seq_len = x.shape
    num, out = W.shape
    assert seq_len == 2048 and num == 2048 and out == 8
    wr = W.reshape(128, 128)
    wext = pl.pallas_call(
        _build_wext_kernel,
        out_shape=jax.ShapeDtypeStruct((256, 128), jnp.float32),
    )(wr).reshape(32768)

    mesh = plsc.VectorSubcoreMesh(core_axis_name="c", subcore_axis_name="s")
    expand = functools.partial(
        pl.kernel,
        mesh=mesh,
        out_type=jax.ShapeDtypeStruct((33554432,), jnp.float32),
        scratch_types=[
            pltpu.VMEM((32768,), jnp.float32),
            pltpu.SemaphoreType.DMA,
        ],
    )(_sc_expand_kernel)
    res = expand(wext)
    emb = res.reshape(1, out, seq_len, seq_len)
    if bs > 1:
        emb = jnp.tile(emb, (bs, 1, 1, 1))
    return emb


# speedup vs baseline: 4.3026x; 4.3026x over previous
import jax
import jax.numpy as jnp
from jax.experimental import pallas as pl
from jax.experimental.pallas import tpu as pltpu


def _posemb_kernel(wr_ref, out_ref, t_ref):
    c = pl.program_id(0)
    h = pl.program_id(1)

    @pl.when((c == 0) & (h == 0))
    def _build_tables():
        f32 = jnp.float32
        w = wr_ref[:]
        ri = jax.lax.broadcasted_iota(jnp.int32, (128, 128), 0)
        ci = jax.lax.broadcasted_iota(jnp.int32, (128, 128), 1)

        def dot(a, b):
            return jax.lax.dot(a, b, preferred_element_type=f32,
                               precision=jax.lax.Precision.HIGHEST)

        perm = (ri == (8 * (15 - ci // 8) + ci % 8)).astype(f32)
        flip = ((ri + ci) == 127).astype(f32)
        rev = dot(flip, dot(w, perm))
        roll8 = (ri == ((ci + 8) % 128)).astype(f32)
        gfw = dot(w, roll8)
        pick = ((ri == 127) & (ci == 0)).astype(f32)
        b0 = dot(pick, gfw)
        low = jnp.where((ri < 127) | (ci < 120), rev, b0)
        gup = jnp.concatenate([gfw[1:], jnp.zeros((1, 128), f32)], axis=0)
        high = jnp.where(ci < 120, gfw, gup)
        wext = jnp.concatenate([low, high], axis=0)

        lane256 = jax.lax.broadcasted_iota(jnp.int32, (256, 128), 1)
        for t in range(16):
            sh = (15 - t) * 8
            rollm = (ri == ((ci + sh) % 128)).astype(f32)
            rolled = dot(wext, rollm)
            rollup = jnp.concatenate(
                [rolled[1:], jnp.zeros((1, 128), f32)], axis=0)
            t_ref[t] = jnp.where(lane256 < (128 - sh), rolled, rollup)

    for dh in range(8):
        r0 = 15 - (8 * h + dh) + 16 * (7 - c)
        out_ref[0, 0, pl.ds(128 * dh, 128)] = (
            t_ref[:, pl.ds(r0, 128), :].reshape(128, 2048))


def kernel(x, W):
    bs, _, seq_len = x.shape
    num, out = W.shape
    assert seq_len == 2048 and num == 2048 and out == 8
    wr = W.reshape(128, 128)
    emb = pl.pallas_call(
        _posemb_kernel,
        grid=(8, 2),
        in_specs=[pl.BlockSpec((128, 128), lambda c, h: (0, 0))],
        out_specs=pl.BlockSpec((1, 1, 1024, 2048), lambda c, h: (0, c, h, 0)),
        out_shape=jax.ShapeDtypeStruct((1, 8, 2048, 2048), jnp.float32),
        scratch_shapes=[
            pltpu.VMEM((16, 256, 128), jnp.float32),
        ],
    )(wr)
    if bs > 1:
        emb = jnp.tile(emb, (bs, 1, 1, 1))
    return emb
